# baseline (speedup 1.0000x reference)
import functools

import jax
import jax.numpy as jnp
from jax import lax
from jax.experimental import pallas as pl
from jax.experimental.pallas import tpu as pltpu
from jax.experimental.pallas import tpu_sc as plsc

_GAMMA = 12.0
_D = 64
_B = 4096
_NEG = 128
_NC = 2
_NS = 16
_NW = _NC * _NS
_BPW = _B // _NW
_L = 16


def _body(hidx_hbm, ridx_hbm, tp_hbm, ent_hbm, rel_hbm, out_hbm,
          tidx_v, hidx_v, ridx_v, hr_v, rel_v,
          tbuf0, tbuf1, tbuf2, tbuf3, score_v, sbuf,
          sem0, sem1, sem2, sem3):
    wid = lax.axis_index("s") * _NC + lax.axis_index("c")
    base = wid * _BPW

    pltpu.sync_copy(hidx_hbm.at[pl.ds(base, _BPW)], hidx_v)
    pltpu.sync_copy(ridx_hbm.at[pl.ds(base, _BPW)], ridx_v)
    pltpu.sync_copy(tp_hbm.at[pl.ds(base * _NEG, _BPW * _NEG)], tidx_v)

    lanes = lax.iota(jnp.int32, _L)

    pltpu.async_copy(ent_hbm.at[hidx_v], hr_v, sem0).wait()
    pltpu.async_copy(rel_hbm.at[ridx_v], rel_v, sem0).wait()

    def hr_body(b, carry):
        for j in range(_D // _L):
            sl = pl.ds(j * _L, _L)
            hr_v[b, sl] = hr_v[b, sl] + rel_v[b, sl]
        return carry
    lax.fori_loop(0, _BPW, hr_body, 0, unroll=4)

    _P = _L + 1

    def compute_b(b, tbuf, off, sbuf):
        hr0 = hr_v[b, pl.ds(0, _L)]
        hr1 = hr_v[b, pl.ds(_L, _L)]
        hr2 = hr_v[b, pl.ds(2 * _L, _L)]
        hr3 = hr_v[b, pl.ds(3 * _L, _L)]

        def diffs(n):
            t0 = tbuf[off + n, pl.ds(0, _L)]
            t1 = tbuf[off + n, pl.ds(_L, _L)]
            t2 = tbuf[off + n, pl.ds(2 * _L, _L)]
            t3 = tbuf[off + n, pl.ds(3 * _L, _L)]
            return (jnp.abs(hr0 - t0) + jnp.abs(hr1 - t1)
                    + jnp.abs(hr2 - t2) + jnp.abs(hr3 - t3))

        def p1(n, idxv):
            plsc.store_scatter(sbuf, [idxv], diffs(n))
            return idxv + _P
        lax.fori_loop(0, _NEG, p1, lanes, unroll=4)

        lanes_p = lanes * _P
        for g in range(_NEG // _L):
            acc = jnp.full((_L,), _GAMMA, jnp.float32)
            base = lanes_p + (g * (_L * _P))
            for l in range(_L):
                acc = acc - plsc.load_gather(sbuf, [base + l])
            score_v[b, pl.ds(g * _L, _L)] = acc

    tbufs = (tbuf0, tbuf1, tbuf2, tbuf3)
    sems = (sem0, sem1, sem2, sem3)
    npair = _BPW // 2

    def gidx(p):
        return tidx_v.at[pl.ds(p * 2 * _NEG, 2 * _NEG)]

    for r in range(3):
        pltpu.async_copy(ent_hbm.at[gidx(r)], tbufs[r], sems[r])

    def outer(i, carry):
        for j in range(4):
            p = 4 * i + j
            nxt = (j + 3) % 4

            @pl.when(p + 3 < npair)
            def _():
                pltpu.async_copy(
                    ent_hbm.at[gidx(p + 3)], tbufs[nxt], sems[nxt])
            pltpu.make_async_copy(
                ent_hbm.at[gidx(p)], tbufs[j], sems[j]).wait()
            compute_b(2 * p, tbufs[j], 0, sbuf)
            compute_b(2 * p + 1, tbufs[j], _NEG, sbuf)
        return carry
    lax.fori_loop(0, npair // 4, outer, 0)

    pltpu.sync_copy(score_v, out_hbm.at[pl.ds(base, _BPW)])


@functools.partial(
    pl.kernel,
    mesh=plsc.VectorSubcoreMesh(core_axis_name="c", subcore_axis_name="s"),
    out_type=jax.ShapeDtypeStruct((_B, _NEG), jnp.float32),
    compiler_params=pltpu.CompilerParams(
        needs_layout_passes=False, use_tc_tiling_on_sc=False),
    scratch_types=[
        pltpu.VMEM((_BPW * _NEG,), jnp.int32),
        pltpu.VMEM((_BPW,), jnp.int32),
        pltpu.VMEM((_BPW,), jnp.int32),
        pltpu.VMEM((_BPW, _D), jnp.float32),
        pltpu.VMEM((_BPW, _D), jnp.float32),
        pltpu.VMEM((2 * _NEG, _D), jnp.float32),
        pltpu.VMEM((2 * _NEG, _D), jnp.float32),
        pltpu.VMEM((2 * _NEG, _D), jnp.float32),
        pltpu.VMEM((2 * _NEG, _D), jnp.float32),
        pltpu.VMEM((_BPW, _NEG), jnp.float32),
        pltpu.VMEM((_NEG * (_L + 1),), jnp.float32),
        pltpu.SemaphoreType.DMA,
        pltpu.SemaphoreType.DMA,
        pltpu.SemaphoreType.DMA,
        pltpu.SemaphoreType.DMA,
    ],
)
def _kge_score(hidx, ridx, tp, ent, rel, out, *scratch):
    _body(hidx, ridx, tp, ent, rel, out, *scratch)


def kernel(head_part, tail_part, entity_embedding, relation_embedding):
    hp = head_part.astype(jnp.int32)
    ent_pad = jnp.pad(entity_embedding, ((0, 0), (0, _D)))
    ent_v = ent_pad.reshape(2 * entity_embedding.shape[0],
                            entity_embedding.shape[1])
    return _kge_score(hp[:, 0] * 2, hp[:, 1],
                      tail_part.astype(jnp.int32).reshape(-1) * 2,
                      ent_v, relation_embedding)


# speedup vs baseline: 1.0224x; 1.0224x over previous
import functools

import jax
import jax.numpy as jnp
from jax import lax
from jax.experimental import pallas as pl
from jax.experimental.pallas import tpu as pltpu
from jax.experimental.pallas import tpu_sc as plsc

_GAMMA = 12.0
_D = 64
_B = 4096
_NEG = 128
_NC = 2
_NS = 16
_NW = _NC * _NS
_BPW = _B // _NW
_L = 16


def _body(hidx_hbm, ridx_hbm, tp_hbm, ent_hbm, rel_hbm, out_hbm,
          tidx_v, hidx_v, ridx_v, hr_v, rel_v,
          tbuf0, tbuf1, tbuf2, tbuf3, score_v,
          sem0, sem1, sem2, sem3):
    wid = lax.axis_index("s") * _NC + lax.axis_index("c")
    base = wid * _BPW

    pltpu.sync_copy(hidx_hbm.at[pl.ds(base, _BPW)], hidx_v)
    pltpu.sync_copy(ridx_hbm.at[pl.ds(base, _BPW)], ridx_v)
    pltpu.sync_copy(tp_hbm.at[pl.ds(base * _NEG, _BPW * _NEG)], tidx_v)

    lanes = lax.iota(jnp.int32, _L)

    pltpu.async_copy(ent_hbm.at[hidx_v], hr_v, sem0).wait()
    pltpu.async_copy(rel_hbm.at[ridx_v], rel_v, sem0).wait()

    def hr_body(b, carry):
        for j in range(_D // _L):
            sl = pl.ds(j * _L, _L)
            hr_v[b, sl] = hr_v[b, sl] + rel_v[b, sl]
        return carry
    lax.fori_loop(0, _BPW, hr_body, 0, unroll=4)

    mask7 = lanes == (_L // 2 - 1)
    mask15 = lanes == (_L - 1)
    m715 = mask7 | mask15
    lane_lo = lanes < (_L // 2)
    is15 = mask15.astype(jnp.int32)

    def compute_b(b, tbuf):
        hr0 = hr_v[b, pl.ds(0, _L)]
        hr1 = hr_v[b, pl.ds(_L, _L)]
        hr2 = hr_v[b, pl.ds(2 * _L, _L)]
        hr3 = hr_v[b, pl.ds(3 * _L, _L)]
        bsplat = jnp.full((_L,), b, jnp.int32)

        def diffs(n):
            t0 = tbuf[n, pl.ds(0, _L)]
            t1 = tbuf[n, pl.ds(_L, _L)]
            t2 = tbuf[n, pl.ds(2 * _L, _L)]
            t3 = tbuf[n, pl.ds(3 * _L, _L)]
            return (jnp.abs(hr0 - t0) + jnp.abs(hr1 - t1)
                    + jnp.abs(hr2 - t2) + jnp.abs(hr3 - t3))

        def nbody(k, carry):
            n0 = 2 * k
            fa = diffs(n0)
            fb = diffs(n0 + 1)
            fa = fa + jnp.flip(fa)
            fb = fb + jnp.flip(fb)
            half = _GAMMA / (_L // 2)
            u = jnp.where(lane_lo, half - fa, half - fb)
            c = plsc.cumsum(u)
            n_idx = jnp.full((_L,), n0, jnp.int32) + is15
            plsc.store_scatter(score_v, [bsplat, n_idx], c, mask=m715)
            n1splat = jnp.full((_L,), n0 + 1, jnp.int32)
            plsc.addupdate_scatter(score_v, [bsplat, n1splat], -c, mask=mask7)
            return carry
        lax.fori_loop(0, _NEG // 2, nbody, 0, unroll=4)

    tbufs = (tbuf0, tbuf1, tbuf2, tbuf3)
    sems = (sem0, sem1, sem2, sem3)

    def gidx(p):
        return tidx_v.at[pl.ds(p * _NEG, _NEG)]

    for r in range(3):
        pltpu.async_copy(ent_hbm.at[gidx(r)], tbufs[r], sems[r])

    def outer(i, carry):
        for j in range(4):
            b = 4 * i + j
            nxt = (j + 3) % 4

            @pl.when(b + 3 < _BPW)
            def _():
                pltpu.async_copy(
                    ent_hbm.at[gidx(b + 3)], tbufs[nxt], sems[nxt])
            pltpu.make_async_copy(
                ent_hbm.at[gidx(b)], tbufs[j], sems[j]).wait()
            compute_b(b, tbufs[j])
        return carry
    lax.fori_loop(0, _BPW // 4, outer, 0)

    pltpu.sync_copy(score_v, out_hbm.at[pl.ds(base, _BPW)])


@functools.partial(
    pl.kernel,
    mesh=plsc.VectorSubcoreMesh(core_axis_name="c", subcore_axis_name="s"),
    out_type=jax.ShapeDtypeStruct((_B, _NEG), jnp.float32),
    compiler_params=pltpu.CompilerParams(
        needs_layout_passes=False, use_tc_tiling_on_sc=False),
    scratch_types=[
        pltpu.VMEM((_BPW * _NEG,), jnp.int32),
        pltpu.VMEM((_BPW,), jnp.int32),
        pltpu.VMEM((_BPW,), jnp.int32),
        pltpu.VMEM((_BPW, _D), jnp.float32),
        pltpu.VMEM((_BPW, _D), jnp.float32),
        pltpu.VMEM((_NEG, _D), jnp.float32),
        pltpu.VMEM((_NEG, _D), jnp.float32),
        pltpu.VMEM((_NEG, _D), jnp.float32),
        pltpu.VMEM((_NEG, _D), jnp.float32),
        pltpu.VMEM((_BPW, _NEG), jnp.float32),
        pltpu.SemaphoreType.DMA,
        pltpu.SemaphoreType.DMA,
        pltpu.SemaphoreType.DMA,
        pltpu.SemaphoreType.DMA,
    ],
)
def _kge_score(hidx, ridx, tp, ent, rel, out, *scratch):
    _body(hidx, ridx, tp, ent, rel, out, *scratch)


def kernel(head_part, tail_part, entity_embedding, relation_embedding):
    hp = head_part.astype(jnp.int32)
    ent_pad = jnp.pad(entity_embedding, ((0, 0), (0, _D)))
    ent_v = ent_pad.reshape(2 * entity_embedding.shape[0],
                            entity_embedding.shape[1])
    return _kge_score(hp[:, 0] * 2, hp[:, 1],
                      tail_part.astype(jnp.int32).reshape(-1) * 2,
                      ent_v, relation_embedding)
